# Initial kernel scaffold; baseline (speedup 1.0000x reference)
import jax
import jax.numpy as jnp
from jax.experimental import pallas as pl


def kernel(x, edge_index, batch_index, W1, b1, W2, b2, W3, b3, Wout, bout):
    raise NotImplementedError("write your pallas kernel here")


# speedup vs baseline: 25.4560x; 25.4560x over previous
import functools

import jax
import jax.numpy as jnp
from jax import lax
from jax.experimental import pallas as pl
from jax.experimental.pallas import tpu as pltpu
from jax.experimental.pallas import tpu_sc as plsc

N = 10000
E = 320000
NUM_GRAPHS = 128
NC = 2
NS = 16
NW = NC * NS
CHUNK = 128
CPT = 80
E_PAD = NW * CPT * CHUNK
N_PAD = 10240
RPW = N_PAD // NS
NPW = N_PAD // NW

_mesh = plsc.VectorSubcoreMesh(core_axis_name="c", subcore_axis_name="s")
_sc_params = pltpu.CompilerParams(use_tc_tiling_on_sc=False)


def _wid():
    return lax.axis_index("c") * NS + lax.axis_index("s")


@functools.partial(
    pl.kernel,
    out_type=jax.ShapeDtypeStruct((NC, N_PAD), jnp.float32),
    mesh=_mesh,
    compiler_params=_sc_params,
    scratch_types=[
        pltpu.VMEM((CPT, CHUNK), jnp.int32),
        pltpu.VMEM((CHUNK,), jnp.float32),
        pltpu.VMEM_SHARED((N_PAD,), jnp.float32),
    ],
)
def _deg_kernel(dst_hbm, zeros_hbm, out_hbm, dst_v, ones_v, acc_sh):
    cid = lax.axis_index("c")
    sid = lax.axis_index("s")
    wid = cid * NS + sid
    pltpu.sync_copy(zeros_hbm.at[pl.ds(sid * RPW, RPW)],
                    acc_sh.at[pl.ds(sid * RPW, RPW)])
    pltpu.sync_copy(dst_hbm.at[pl.ds(wid * CPT, CPT)], dst_v)
    for i in range(CHUNK // 16):
        ones_v[pl.ds(16 * i, 16)] = jnp.full((16,), 1.0, jnp.float32)
    plsc.subcore_barrier()

    def body(j, carry):
        pltpu.sync_copy(ones_v, acc_sh.at[dst_v.at[j]], add=True)
        return carry

    lax.fori_loop(0, CPT, body, 0)
    plsc.subcore_barrier()
    pltpu.sync_copy(acc_sh.at[pl.ds(sid * RPW, RPW)],
                    out_hbm.at[cid, pl.ds(sid * RPW, RPW)])


def _make_edge_scatter(F):
    @functools.partial(
        pl.kernel,
        out_type=jax.ShapeDtypeStruct((NC, N_PAD, F), jnp.float32),
        mesh=_mesh,
        compiler_params=_sc_params,
        scratch_types=[
            pltpu.VMEM((CPT, CHUNK), jnp.int32),
            pltpu.VMEM((CPT, CHUNK), jnp.int32),
            pltpu.VMEM((CHUNK, F), jnp.float32),
            pltpu.VMEM_SHARED((N_PAD, F), jnp.float32),
            pltpu.SemaphoreType.DMA,
        ],
    )
    def k(g_hbm, src_hbm, dst_hbm, zeros_hbm, out_hbm,
          src_v, dst_v, rows_v, acc_sh, sem):
        cid = lax.axis_index("c")
        sid = lax.axis_index("s")
        wid = cid * NS + sid
        pltpu.sync_copy(zeros_hbm.at[pl.ds(sid * RPW, RPW)],
                        acc_sh.at[pl.ds(sid * RPW, RPW)])
        pltpu.sync_copy(src_hbm.at[pl.ds(wid * CPT, CPT)], src_v)
        pltpu.sync_copy(dst_hbm.at[pl.ds(wid * CPT, CPT)], dst_v)
        plsc.subcore_barrier()

        def body(j, carry):
            pltpu.async_copy(g_hbm.at[src_v.at[j]], rows_v, sem).wait()
            pltpu.sync_copy(rows_v, acc_sh.at[dst_v.at[j]], add=True)
            return carry

        lax.fori_loop(0, CPT, body, 0)
        plsc.subcore_barrier()
        pltpu.sync_copy(acc_sh.at[pl.ds(sid * RPW, RPW)],
                        out_hbm.at[cid, pl.ds(sid * RPW, RPW)])

    return k


_scatter = {F: _make_edge_scatter(F) for F in (32, 48, 64)}


@functools.partial(
    pl.kernel,
    out_type=jax.ShapeDtypeStruct((NW, NUM_GRAPHS, 64), jnp.float32),
    mesh=_mesh,
    compiler_params=_sc_params,
    scratch_types=[
        pltpu.VMEM((NPW + 16,), jnp.int32),
        pltpu.VMEM((NPW, 64), jnp.float32),
        pltpu.VMEM((NUM_GRAPHS, 64), jnp.float32),
    ],
)
def _maxpool_kernel(h_hbm, bidx_hbm, out_hbm, bidx_v, hbuf, macc):
    wid = _wid()
    base = wid * NPW
    pltpu.sync_copy(bidx_hbm.at[pl.ds(base, NPW)], bidx_v.at[pl.ds(0, NPW)])
    pltpu.sync_copy(h_hbm.at[pl.ds(base, NPW)], hbuf)

    def init(i, carry):
        for j in range(4):
            macc[i, pl.ds(16 * j, 16)] = jnp.full((16,), -jnp.inf, jnp.float32)
        return carry

    lax.fori_loop(0, NUM_GRAPHS, init, 0)

    def body(i, carry):
        b = bidx_v[pl.ds(i, 16)][0]
        for j in range(4):
            sl = pl.ds(16 * j, 16)
            macc[b, sl] = jnp.maximum(macc[b, sl], hbuf[i, sl])
        return carry

    lax.fori_loop(0, NPW, body, 0)
    pltpu.sync_copy(macc, out_hbm.at[wid])


def _tc1_body(x_ref, w_ref, degp_ref, g_ref, dinv_ref):
    dp = degp_ref[...]
    deg = 1.0 + dp[0, :N] + dp[1, :N]
    dinv = lax.rsqrt(deg)[:, None]
    dinv_ref[...] = dinv
    g_ref[...] = dinv * jnp.dot(x_ref[...], w_ref[...],
                                preferred_element_type=jnp.float32)


def _tc1(x, W1, degp):
    return pl.pallas_call(
        _tc1_body,
        out_shape=(jax.ShapeDtypeStruct((N, W1.shape[1]), jnp.float32),
                   jax.ShapeDtypeStruct((N, 1), jnp.float32)),
    )(x, W1, degp)


def _tc_mid_body(sp_ref, g_ref, dinv_ref, b_ref, w_ref, gn_ref):
    sp = sp_ref[...]
    s = sp[0, :N] + sp[1, :N]
    h = jnp.maximum(dinv_ref[...] * (s + g_ref[...]) + b_ref[...], 0.0)
    gn_ref[...] = dinv_ref[...] * jnp.dot(h, w_ref[...],
                                          preferred_element_type=jnp.float32)


def _tc_mid(sp, g, dinv, b, Wn):
    return pl.pallas_call(
        _tc_mid_body,
        out_shape=jax.ShapeDtypeStruct((N, Wn.shape[1]), jnp.float32),
    )(sp, g, dinv, b.reshape(1, -1), Wn)


def _tc_last_body(sp_ref, g_ref, dinv_ref, b_ref, h_ref):
    sp = sp_ref[...]
    s = sp[0, :N] + sp[1, :N]
    h_ref[...] = jnp.maximum(dinv_ref[...] * (s + g_ref[...]) + b_ref[...], 0.0)


def _tc_last(sp, g, dinv, b):
    return pl.pallas_call(
        _tc_last_body,
        out_shape=jax.ShapeDtypeStruct((N, g.shape[1]), jnp.float32),
    )(sp, g, dinv, b.reshape(1, -1))


def _tc_final_body(maxp_ref, h_ref, bidx_ref, wout_ref, bout_ref, out_ref):
    gmp = jnp.max(maxp_ref[...], axis=0)
    oh = (bidx_ref[...] ==
          lax.broadcasted_iota(jnp.int32, (1, NUM_GRAPHS), 1)
          ).astype(jnp.float32)
    s = lax.dot_general(oh, h_ref[...], (((0,), (0,)), ((), ())),
                        preferred_element_type=jnp.float32)
    cnt = jnp.sum(oh, axis=0)[:, None]
    gap = s / jnp.maximum(cnt, 1.0)
    hidden = jnp.concatenate([gmp, gap], axis=1)
    out_ref[...] = jnp.dot(hidden, wout_ref[...],
                           preferred_element_type=jnp.float32) + bout_ref[...]


def _tc_final(maxp, h3, bidx, Wout, bout):
    return pl.pallas_call(
        _tc_final_body,
        out_shape=jax.ShapeDtypeStruct((NUM_GRAPHS, Wout.shape[1]), jnp.float32),
    )(maxp, h3, bidx.reshape(-1, 1), Wout, bout.reshape(1, -1))


def kernel(x, edge_index, batch_index, W1, b1, W2, b2, W3, b3, Wout, bout):
    src, dst = edge_index[0], edge_index[1]
    pad = E_PAD - E
    src_p = jnp.concatenate([src, (jnp.arange(pad, dtype=jnp.int32) * 97) % N])
    dst_p = jnp.concatenate(
        [dst, N + (jnp.arange(pad, dtype=jnp.int32) % (N_PAD - N))])
    src2d = src_p.reshape(NW * CPT, CHUNK)
    dst2d = dst_p.reshape(NW * CPT, CHUNK)

    zeros1 = jnp.zeros((N_PAD,), jnp.float32)
    degp = _deg_kernel(dst2d, zeros1)

    g1, dinv = _tc1(x, W1, degp)
    s1 = _scatter[32](g1, src2d, dst2d, jnp.zeros((N_PAD, 32), jnp.float32))
    g2 = _tc_mid(s1, g1, dinv, b1, W2)
    s2 = _scatter[48](g2, src2d, dst2d, jnp.zeros((N_PAD, 48), jnp.float32))
    g3 = _tc_mid(s2, g2, dinv, b2, W3)
    s3 = _scatter[64](g3, src2d, dst2d, jnp.zeros((N_PAD, 64), jnp.float32))
    h3 = _tc_last(s3, g3, dinv, b3)

    h3p = jnp.concatenate(
        [h3, jnp.full((N_PAD - N, 64), -jnp.inf, jnp.float32)], axis=0)
    bidxp = jnp.concatenate(
        [batch_index, jnp.full((N_PAD - N,), NUM_GRAPHS - 1, jnp.int32)])
    maxp = _maxpool_kernel(h3p, bidxp)

    return _tc_final(maxp, h3, batch_index, Wout, bout)
